# Initial kernel scaffold; baseline (speedup 1.0000x reference)
import jax
import jax.numpy as jnp
from jax.experimental import pallas as pl


def kernel(x, edge_index, batch, W1, b1, W2, b2, W3, b3, Wfc, bfc):
    raise NotImplementedError("write your pallas kernel here")


# speedup vs baseline: 10.9933x; 10.9933x over previous
import functools

import jax
import jax.numpy as jnp
from jax import lax
from jax.experimental import pallas as pl
from jax.experimental.pallas import tpu as pltpu
from jax.experimental.pallas import tpu_sc as plsc

N = 10000
NP = 10240
E = 320000
G = 64
NC = 2
NS = 16
ROWS_PER_TILE = NP // NS
CHUNK = 128
RB = NP // 128


def _sc_mesh():
    return plsc.VectorSubcoreMesh(
        core_axis_name="c", subcore_axis_name="s",
        num_cores=NC, num_subcores=NS)


_EPT32 = E // (NC * NS)
_DEG_CHUNKS = _EPT32 // CHUNK
_DEG_TAIL = _EPT32 - _DEG_CHUNKS * CHUNK


def _degree_fn():
    @functools.partial(
        pl.kernel,
        out_type=jax.ShapeDtypeStruct((NC, NP, 16), jnp.float32),
        mesh=_sc_mesh(),
        compiler_params=pltpu.CompilerParams(use_tc_tiling_on_sc=False),
        scratch_types=[
            pltpu.VMEM_SHARED((NP, 16), jnp.float32),
            pltpu.VMEM((CHUNK, 16), jnp.float32),
            pltpu.VMEM((CHUNK,), jnp.int32),
            pltpu.VMEM((_DEG_TAIL, 16), jnp.float32),
            pltpu.VMEM((_DEG_TAIL,), jnp.int32),
        ],
    )
    def deg_kernel(dst_hbm, zeros_hbm, ones_hbm, out_hbm,
                   deg_sp, ones_v, idx_v, ones_t, idx_t):
        c = lax.axis_index("c")
        s = lax.axis_index("s")
        r0 = s * ROWS_PER_TILE
        pltpu.sync_copy(zeros_hbm.at[pl.ds(r0, ROWS_PER_TILE)],
                        deg_sp.at[pl.ds(r0, ROWS_PER_TILE)])
        pltpu.sync_copy(ones_hbm, ones_v)
        pltpu.sync_copy(ones_hbm.at[pl.ds(0, _DEG_TAIL)], ones_t)
        plsc.subcore_barrier()
        base = (c * NS + s) * _EPT32

        def chunk(j, carry):
            off = pl.multiple_of(base + j * CHUNK, 8)
            pltpu.sync_copy(dst_hbm.at[pl.ds(off, CHUNK)], idx_v)
            pltpu.sync_copy(ones_v, deg_sp.at[idx_v], add=True)
            return carry

        lax.fori_loop(0, _DEG_CHUNKS, chunk, 0)
        toff = pl.multiple_of(base + _DEG_CHUNKS * CHUNK, 8)
        pltpu.sync_copy(dst_hbm.at[pl.ds(toff, _DEG_TAIL)], idx_t)
        pltpu.sync_copy(ones_t, deg_sp.at[idx_t], add=True)
        plsc.subcore_barrier()
        pltpu.sync_copy(deg_sp.at[pl.ds(r0, ROWS_PER_TILE)],
                        out_hbm.at[c, pl.ds(r0, ROWS_PER_TILE)])

    return deg_kernel


def _make_spmm(dh):
    ept = E // NS
    n_chunks = ept // CHUNK
    tail = ept - n_chunks * CHUNK

    @functools.partial(
        pl.kernel,
        out_type=jax.ShapeDtypeStruct((NC * NP, dh), jnp.float32),
        mesh=_sc_mesh(),
        compiler_params=pltpu.CompilerParams(use_tc_tiling_on_sc=False),
        scratch_types=[
            pltpu.VMEM_SHARED((NP, dh), jnp.float32),
            pltpu.VMEM((CHUNK, dh), jnp.float32),
            pltpu.VMEM((CHUNK,), jnp.int32),
            pltpu.VMEM((CHUNK,), jnp.int32),
            pltpu.VMEM((tail, dh), jnp.float32),
            pltpu.VMEM((tail,), jnp.int32),
            pltpu.VMEM((tail,), jnp.int32),
            pltpu.SemaphoreType.DMA,
        ],
    )
    def spmm(g_hbm, src2_hbm, dst_hbm, out_hbm,
             s_sp, rows_v, src_v, dst_v, rows_t, src_t, dst_t, sem):
        c = lax.axis_index("c")
        s = lax.axis_index("s")
        r0 = s * ROWS_PER_TILE
        tbl = c * NP
        pltpu.sync_copy(g_hbm.at[pl.ds(tbl + r0, ROWS_PER_TILE)],
                        s_sp.at[pl.ds(r0, ROWS_PER_TILE)])
        plsc.subcore_barrier()
        ebase = s * ept

        def chunk(j, carry):
            off = pl.multiple_of(ebase + j * CHUNK, 8)
            pltpu.sync_copy(src2_hbm.at[pl.ds(c * E + off, CHUNK)], src_v)
            pltpu.sync_copy(dst_hbm.at[pl.ds(off, CHUNK)], dst_v)
            pltpu.async_copy(g_hbm.at[src_v], rows_v, sem).wait()
            pltpu.sync_copy(rows_v, s_sp.at[dst_v], add=True)
            return carry

        lax.fori_loop(0, n_chunks, chunk, 0)
        toff = pl.multiple_of(ebase + n_chunks * CHUNK, 8)
        pltpu.sync_copy(src2_hbm.at[pl.ds(c * E + toff, tail)], src_t)
        pltpu.sync_copy(dst_hbm.at[pl.ds(toff, tail)], dst_t)
        pltpu.async_copy(g_hbm.at[src_t], rows_t, sem).wait()
        pltpu.sync_copy(rows_t, s_sp.at[dst_t], add=True)
        plsc.subcore_barrier()
        pltpu.sync_copy(s_sp.at[pl.ds(r0, ROWS_PER_TILE)],
                        out_hbm.at[pl.ds(tbl + r0, ROWS_PER_TILE)])

    return spmm


def _prep_body(deg_ref, x_ref, dinv_ref, g_ref):
    deg = deg_ref[0, :, 0:1] + deg_ref[1, :, 0:1] + 1.0
    dinv = lax.rsqrt(deg)
    dinv_ref[...] = dinv
    g = x_ref[...] * dinv
    g_ref[0] = g[:, 0:64]
    g_ref[1] = g[:, 64:128]


def _prep(deg, x):
    return pl.pallas_call(
        _prep_body,
        grid=(RB,),
        in_specs=[
            pl.BlockSpec((2, 128, 16), lambda r: (0, r, 0)),
            pl.BlockSpec((128, 128), lambda r: (r, 0)),
        ],
        out_specs=[
            pl.BlockSpec((128, 1), lambda r: (r, 0)),
            pl.BlockSpec((2, 128, 64), lambda r: (0, r, 0)),
        ],
        out_shape=[
            jax.ShapeDtypeStruct((NP, 1), jnp.float32),
            jax.ShapeDtypeStruct((2, NP, 64), jnp.float32),
        ],
        compiler_params=pltpu.CompilerParams(
            dimension_semantics=("arbitrary",)),
    )(deg, x)


def _make_layer(d_in, d_out):
    dh_in = d_in // 2
    dh_out = d_out // 2

    def body(s_ref, dinv_ref, w_ref, b_ref, out_ref):
        t = jnp.dot(s_ref[0], w_ref[0:dh_in, :],
                    preferred_element_type=jnp.float32)
        t = t + jnp.dot(s_ref[1], w_ref[dh_in:d_in, :],
                        preferred_element_type=jnp.float32)
        act = jnp.maximum(dinv_ref[...] * t + b_ref[...], 0.0)
        g = act * dinv_ref[...]
        out_ref[0] = g[:, 0:dh_out]
        out_ref[1] = g[:, dh_out:d_out]

    def layer(s, dinv, w, b):
        return pl.pallas_call(
            body,
            grid=(RB,),
            in_specs=[
                pl.BlockSpec((2, 128, dh_in), lambda r: (0, r, 0)),
                pl.BlockSpec((128, 1), lambda r: (r, 0)),
                pl.BlockSpec((d_in, d_out), lambda r: (0, 0)),
                pl.BlockSpec((1, d_out), lambda r: (0, 0)),
            ],
            out_specs=pl.BlockSpec((2, 128, dh_out), lambda r: (0, r, 0)),
            out_shape=jax.ShapeDtypeStruct((2, NP, dh_out), jnp.float32),
            compiler_params=pltpu.CompilerParams(
                dimension_semantics=("arbitrary",)),
        )(s, dinv, w, b)

    return layer


def _pool_body(s_ref, dinv_ref, batch_ref, w_ref, b_ref, wfc_ref, bfc_ref,
               out_ref, acc_ref):
    r = pl.program_id(0)

    @pl.when(r == 0)
    def _init():
        acc_ref[...] = jnp.full((G, 512), -jnp.inf, jnp.float32)

    t = jnp.dot(s_ref[0], w_ref[0:128, :], preferred_element_type=jnp.float32)
    t = t + jnp.dot(s_ref[1], w_ref[128:256, :],
                    preferred_element_type=jnp.float32)
    h = jnp.maximum(dinv_ref[...] * t + b_ref[...], 0.0)
    rid = lax.broadcasted_iota(jnp.int32, (128, 1), 0) + r * 128
    hm = jnp.where(rid < N, h, -jnp.inf)
    g_lo = jnp.clip(batch_ref[0, 0], 0, G - 1)
    g_hi = jnp.clip(batch_ref[127, 0], g_lo, G - 1)
    gids = lax.broadcasted_iota(jnp.int32, (G, 1), 0)

    def body(g, carry):
        sel = jnp.where(batch_ref[...] == g, hm, -jnp.inf)
        colmax = jnp.max(sel, axis=0, keepdims=True)
        acc_ref[...] = jnp.where(gids == g,
                                 jnp.maximum(acc_ref[...], colmax),
                                 acc_ref[...])
        return carry

    lax.fori_loop(g_lo, g_hi + 1, body, 0)

    @pl.when(r == RB - 1)
    def _fin():
        out_ref[...] = jnp.dot(acc_ref[...], wfc_ref[...],
                               preferred_element_type=jnp.float32) + bfc_ref[...]


def _pool(s3, dinv, batch2, w3, b3, wfc, bfc):
    return pl.pallas_call(
        _pool_body,
        grid=(RB,),
        in_specs=[
            pl.BlockSpec((2, 128, 128), lambda r: (0, r, 0)),
            pl.BlockSpec((128, 1), lambda r: (r, 0)),
            pl.BlockSpec((128, 1), lambda r: (r, 0)),
            pl.BlockSpec((256, 512), lambda r: (0, 0)),
            pl.BlockSpec((1, 512), lambda r: (0, 0)),
            pl.BlockSpec((512, 128), lambda r: (0, 0)),
            pl.BlockSpec((1, 128), lambda r: (0, 0)),
        ],
        out_specs=pl.BlockSpec((G, 128), lambda r: (0, 0)),
        out_shape=jax.ShapeDtypeStruct((G, 128), jnp.float32),
        scratch_shapes=[pltpu.VMEM((G, 512), jnp.float32)],
        compiler_params=pltpu.CompilerParams(
            dimension_semantics=("arbitrary",)),
    )(s3, dinv, batch2, w3, b3, wfc, bfc)


_degree = _degree_fn()
_spmm64 = _make_spmm(64)
_spmm128 = _make_spmm(128)
_layer1 = _make_layer(128, 128)
_layer2 = _make_layer(128, 256)


def kernel(x, edge_index, batch, W1, b1, W2, b2, W3, b3, Wfc, bfc):
    src = edge_index[0].astype(jnp.int32)
    dst = edge_index[1].astype(jnp.int32)
    src2 = jnp.concatenate([src, src + NP])
    zeros16 = jnp.zeros((NP, 16), jnp.float32)
    ones16 = jnp.ones((CHUNK, 16), jnp.float32)
    x_pad = jnp.concatenate(
        [x, jnp.zeros((NP - N, x.shape[1]), jnp.float32)])
    batch2 = jnp.concatenate(
        [batch.astype(jnp.int32),
         jnp.full((NP - N,), G - 1, jnp.int32)]).reshape(NP, 1)

    deg = _degree(dst, zeros16, ones16)
    dinv, g1 = _prep(deg, x_pad)
    s1 = _spmm64(g1.reshape(2 * NP, 64), src2, dst).reshape(2, NP, 64)
    g2 = _layer1(s1, dinv, W1, b1.reshape(1, -1))
    s2 = _spmm64(g2.reshape(2 * NP, 64), src2, dst).reshape(2, NP, 64)
    g3 = _layer2(s2, dinv, W2, b2.reshape(1, -1))
    s3 = _spmm128(g3.reshape(2 * NP, 128), src2, dst).reshape(2, NP, 128)
    out = _pool(s3, dinv, batch2, W3, b3.reshape(1, -1),
                Wfc, bfc.reshape(1, -1))
    return out
